# Initial kernel scaffold; baseline (speedup 1.0000x reference)
import jax
import jax.numpy as jnp
from jax.experimental import pallas as pl


def kernel(points, features, W0, g0, b0, m0, v0, W1, g1, b1, m1, v1, W2, g2, b2, m2, v2, Wsc, gsc, bsc, msc, vsc):
    raise NotImplementedError("write your pallas kernel here")


# speedup vs baseline: 13.4603x; 13.4603x over previous
import functools

import jax
import jax.numpy as jnp
from jax import lax
from jax.experimental import pallas as pl
from jax.experimental.pallas import tpu as pltpu
from jax.experimental.pallas import tpu_sc as plsc

_B, _N, _PD, _C = 8, 2048, 3, 64
_K = 16
_EPS = 1e-3
_BN = 256
_BM = 128
_NW = 32
_R = _B * _N * _K
_PW = _R // _NW
_CHUNK = 128


def _knn_body(pq_ref, pall_ref, f_ref, wg_ref, idx_ref, g_ref):
    A = pq_ref[0]
    P = pall_ref[0]
    F = f_ref[0]
    m = lax.dot_general(A, P, (((1,), (1,)), ((), ())),
                        preferred_element_type=jnp.float32)
    rA = jnp.sum(A * A, axis=1, keepdims=True)
    rB = jnp.sum(P * P, axis=1)
    d = rA - 2.0 * m + rB[None, :]
    d = jnp.maximum(d, 0.0)
    n_base = pl.program_id(1) * _BN
    col = lax.broadcasted_iota(jnp.int32, (_BN, _N), 1)
    rowg = n_base + lax.broadcasted_iota(jnp.int32, (_BN, _N), 0)
    d = jnp.where(col == rowg, jnp.inf, d)
    key = (lax.bitcast_convert_type(d, jnp.int32) & jnp.int32(-2048)) | col
    base = pl.program_id(0) * _N
    cols = []
    for _ in range(_K):
        mn = jnp.min(key, axis=1, keepdims=True)
        cols.append((mn & 2047) + base)
        key = jnp.where(key == mn, jnp.int32(0x7FFFFFFF), key)
    idx_ref[0] = jnp.concatenate(cols, axis=1)
    g = jnp.dot(F, wg_ref[...], preferred_element_type=jnp.float32)
    g_ref[0] = jnp.concatenate([g, jnp.zeros((_BN, _C), jnp.float32)], axis=1)


def _gather_body(g_hbm, idx_hbm, out_hbm, idx_v, rows_v, sem):
    wid = lax.axis_index("s") * 2 + lax.axis_index("c")
    base = wid * _PW

    def step(c, carry):
        off = base + c * _CHUNK
        pltpu.sync_copy(idx_hbm.at[pl.ds(off, _CHUNK)], idx_v)
        pltpu.async_copy(g_hbm.at[idx_v], rows_v, sem).wait()
        pltpu.sync_copy(rows_v, out_hbm.at[pl.ds(off, _CHUNK)])
        return carry

    lax.fori_loop(0, _PW // _CHUNK, step, 0)


def _mlp_body(f_ref, kg_ref, wh_ref, t0_ref, w1_ref, t1_ref, w2_ref, t2_ref,
              wsc_ref, tsc_ref, o_ref):
    F = f_ref[0]
    KG = kg_ref[0][:, :_C]
    H = jnp.dot(F, wh_ref[...], preferred_element_type=jnp.float32) + t0_ref[...]
    S = jnp.dot(F, wsc_ref[...], preferred_element_type=jnp.float32) + tsc_ref[...]
    x0 = KG.reshape(_BM, _K, _C) + H[:, None, :]
    x0 = jnp.maximum(x0, 0.0).reshape(_BM * _K, _C)
    h1 = jnp.maximum(
        jnp.dot(x0, w1_ref[...], preferred_element_type=jnp.float32) + t1_ref[...], 0.0)
    h2 = jnp.maximum(
        jnp.dot(h1, w2_ref[...], preferred_element_type=jnp.float32) + t2_ref[...], 0.0)
    fts = jnp.mean(h2.reshape(_BM, _K, _C), axis=1)
    o_ref[0] = jnp.maximum(S + fts, 0.0)


def _knn_call(pts8, features, Wg):
    return pl.pallas_call(
        _knn_body,
        grid=(_B, _N // _BN),
        in_specs=[
            pl.BlockSpec((1, _BN, 8), lambda b, i: (b, i, 0)),
            pl.BlockSpec((1, _N, 8), lambda b, i: (b, 0, 0)),
            pl.BlockSpec((1, _BN, _C), lambda b, i: (b, i, 0)),
            pl.BlockSpec((_C, _C), lambda b, i: (0, 0)),
        ],
        out_specs=[
            pl.BlockSpec((1, _BN, _K), lambda b, i: (b, i, 0)),
            pl.BlockSpec((1, _BN, 2 * _C), lambda b, i: (b, i, 0)),
        ],
        out_shape=[
            jax.ShapeDtypeStruct((_B, _N, _K), jnp.int32),
            jax.ShapeDtypeStruct((_B, _N, 2 * _C), jnp.float32),
        ],
    )(pts8, pts8, features, Wg)


def _gather_call():
    mesh = plsc.VectorSubcoreMesh(core_axis_name="c", subcore_axis_name="s")
    return pl.kernel(
        _gather_body,
        mesh=mesh,
        out_type=jax.ShapeDtypeStruct((_R, 2 * _C), jnp.float32),
        scratch_types=[
            pltpu.VMEM((_CHUNK,), jnp.int32),
            pltpu.VMEM((_CHUNK, 2 * _C), jnp.float32),
            pltpu.SemaphoreType.DMA,
        ],
    )


def _mlp_call(features, KG, Wh, t0, W1s, t1, W2s, t2, Wscs, tsc):
    wspec = pl.BlockSpec((_C, _C), lambda b, i: (0, 0))
    bspec = pl.BlockSpec((1, _C), lambda b, i: (0, 0))
    return pl.pallas_call(
        _mlp_body,
        grid=(_B, _N // _BM),
        in_specs=[
            pl.BlockSpec((1, _BM, _C), lambda b, i: (b, i, 0)),
            pl.BlockSpec((1, _BM * _K, 2 * _C), lambda b, i: (b, i, 0)),
            wspec, bspec, wspec, bspec, wspec, bspec, wspec, bspec,
        ],
        out_specs=pl.BlockSpec((1, _BM, _C), lambda b, i: (b, i, 0)),
        out_shape=jax.ShapeDtypeStruct((_B, _N, _C), jnp.float32),
    )(features, KG, Wh, t0, W1s, t1, W2s, t2, Wscs, tsc)


def kernel(points, features, W0, g0, b0, m0, v0, W1, g1, b1, m1, v1,
           W2, g2, b2, m2, v2, Wsc, gsc, bsc, msc, vsc):
    s0 = g0 / jnp.sqrt(v0 + _EPS)
    s1 = g1 / jnp.sqrt(v1 + _EPS)
    s2 = g2 / jnp.sqrt(v2 + _EPS)
    ssc = gsc / jnp.sqrt(vsc + _EPS)
    Wg = W0[_C:] * s0[None, :]
    Wh = (W0[:_C] - W0[_C:]) * s0[None, :]
    t0 = (b0 - m0 * s0).reshape(1, _C)
    W1s = W1 * s1[None, :]
    t1 = (b1 - m1 * s1).reshape(1, _C)
    W2s = W2 * s2[None, :]
    t2 = (b2 - m2 * s2).reshape(1, _C)
    Wscs = Wsc * ssc[None, :]
    tsc = (bsc - msc * ssc).reshape(1, _C)

    pts8 = jnp.pad(points, ((0, 0), (0, 0), (0, 8 - _PD)))
    idx, G = _knn_call(pts8, features, Wg)
    KG = _gather_call()(G.reshape(_B * _N, 2 * _C), idx.reshape(_R))
    out = _mlp_call(features, KG.reshape(_B, _N * _K, 2 * _C),
                    Wh, t0, W1s, t1, W2s, t2, Wscs, tsc)
    return out
